# Initial kernel scaffold; baseline (speedup 1.0000x reference)
import jax
import jax.numpy as jnp
from jax.experimental import pallas as pl


def kernel(x, gi, ee):
    raise NotImplementedError("write your pallas kernel here")


# speedup vs baseline: 3.1868x; 3.1868x over previous
import functools

import jax
import jax.numpy as jnp
from jax import lax
from jax.experimental import pallas as pl
from jax.experimental.pallas import tpu as pltpu
from jax.experimental.pallas import tpu_sc as plsc

_ED = 64
_NB = 8
_TOTAL = 1024 * _NB * _NB * _NB
_NW = 32
_PER_W = _TOTAL // _NW
_CHUNK = 128
_NCH = _PER_W // _CHUNK


def _sc_gather(table, gi_flat):
    mesh = plsc.VectorSubcoreMesh(core_axis_name="c", subcore_axis_name="s")

    @functools.partial(
        pl.kernel,
        mesh=mesh,
        out_type=jax.ShapeDtypeStruct((_TOTAL, _ED), jnp.float32),
        scratch_types=[
            pltpu.VMEM((_CHUNK,), jnp.int32),
            pltpu.VMEM((_CHUNK, _ED), jnp.float32),
            pltpu.SemaphoreType.DMA,
        ],
        compiler_params=pltpu.CompilerParams(use_tc_tiling_on_sc=False),
    )
    def k(table_hbm, gi_hbm, out_hbm, idx_v, rows_v, sem):
        wid = lax.axis_index("s") * 2 + lax.axis_index("c")
        base = wid * _PER_W

        def body(g, carry):
            off = pl.multiple_of(base + g * _CHUNK, _CHUNK)
            pltpu.sync_copy(gi_hbm.at[pl.ds(off, _CHUNK)], idx_v)
            pltpu.async_copy(table_hbm.at[idx_v], rows_v, sem).wait()
            pltpu.sync_copy(rows_v, out_hbm.at[pl.ds(off, _CHUNK)])
            return carry

        lax.fori_loop(0, _NCH, body, 0)

    return k(table, gi_flat)


def kernel(x, gi, ee):
    table = jnp.concatenate([ee, x], axis=0)
    gi_flat = gi.reshape(-1).astype(jnp.int32)
    out = _sc_gather(table, gi_flat)
    return out.reshape(gi.shape[0], _NB, _NB, _NB, _ED)


# speedup vs baseline: 3.5823x; 1.1241x over previous
import functools

import jax
import jax.numpy as jnp
from jax import lax
from jax.experimental import pallas as pl
from jax.experimental.pallas import tpu as pltpu
from jax.experimental.pallas import tpu_sc as plsc

_ED = 64
_NB = 8
_TOTAL = 1024 * _NB * _NB * _NB
_NW = 32
_PER_W = _TOTAL // _NW
_GW = 128
_GPB = 2
_CHUNK = _GW * _GPB
_NCH = _PER_W // _CHUNK
_NBUF = 4
_NOUT = _NCH // _NBUF


def _sc_gather(table, gi_tiles):
    mesh = plsc.VectorSubcoreMesh(core_axis_name="c", subcore_axis_name="s")

    @functools.partial(
        pl.kernel,
        mesh=mesh,
        out_type=jax.ShapeDtypeStruct((_TOTAL, _ED), jnp.float32),
        scratch_types=[
            pltpu.VMEM((_NCH, _GPB, _GW), jnp.int32),
            [pltpu.VMEM((_CHUNK, _ED), jnp.float32) for _ in range(_NBUF)],
            [pltpu.SemaphoreType.DMA for _ in range(_NBUF)],
            [pltpu.SemaphoreType.DMA for _ in range(_NBUF)],
            pltpu.SemaphoreType.DMA,
        ],
        compiler_params=pltpu.CompilerParams(use_tc_tiling_on_sc=False),
    )
    def k(table_hbm, gi_hbm, out_hbm, idx_v, rows, gsem, wsem, isem):
        wid = lax.axis_index("s") * 2 + lax.axis_index("c")
        base = wid * _PER_W

        pltpu.async_copy(gi_hbm.at[wid], idx_v, isem).wait()

        def start_gathers(b, c):
            for h in range(_GPB):
                pltpu.async_copy(
                    table_hbm.at[idx_v.at[c, h]],
                    rows[b].at[pl.ds(h * _GW, _GW)],
                    gsem[b],
                )

        def wait_gathers(b, c):
            for h in range(_GPB):
                pltpu.make_async_copy(
                    table_hbm.at[idx_v.at[c, h]],
                    rows[b].at[pl.ds(h * _GW, _GW)],
                    gsem[b],
                ).wait()

        def out_ref(c):
            off = pl.multiple_of(base + c * _CHUNK, _CHUNK)
            return out_hbm.at[pl.ds(off, _CHUNK)]

        for b in range(_NBUF):
            start_gathers(b, b)

        def body(outer, carry):
            for b in range(_NBUF):
                c = outer * _NBUF + b
                wait_gathers(b, c)
                pltpu.async_copy(rows[b], out_ref(c), wsem[b])
            for b in range(_NBUF):
                c = outer * _NBUF + b
                pltpu.make_async_copy(rows[b], out_ref(c), wsem[b]).wait()
                start_gathers(b, c + _NBUF)
            return carry

        lax.fori_loop(0, _NOUT - 1, body, 0)

        for b in range(_NBUF):
            c = (_NOUT - 1) * _NBUF + b
            wait_gathers(b, c)
            pltpu.async_copy(rows[b], out_ref(c), wsem[b])
        for b in range(_NBUF):
            c = (_NOUT - 1) * _NBUF + b
            pltpu.make_async_copy(rows[b], out_ref(c), wsem[b]).wait()

    return k(table, gi_tiles)


def kernel(x, gi, ee):
    table = jnp.concatenate([ee, x], axis=0)
    gi_tiles = gi.reshape(_NW, _NCH, _GPB, _GW).astype(jnp.int32)
    out = _sc_gather(table, gi_tiles)
    return out.reshape(gi.shape[0], _NB, _NB, _NB, _ED)
